# Initial kernel scaffold; baseline (speedup 1.0000x reference)
import jax
import jax.numpy as jnp
from jax.experimental import pallas as pl


def kernel(x, edge_index, W1, b1, W2, b2, Ws, bs):
    raise NotImplementedError("write your pallas kernel here")


# speedup vs baseline: 16.7181x; 16.7181x over previous
import functools

import jax
import jax.numpy as jnp
from jax import lax
from jax.experimental import pallas as pl
from jax.experimental.pallas import tpu as pltpu
from jax.experimental.pallas import tpu_sc as plsc

N_NODES = 10000
N_EDGES = 320000
FDIM = 128

NC = 2
NS = 16
NW = NC * NS
E_PER = N_EDGES // NW
CHUNK = 128
NFULL = E_PER // CHUNK
TAIL = E_PER - NFULL * CHUNK
NPAD = 10240
HSLICE = NPAD // NS
RPT = NPAD // NS

_MESH = plsc.VectorSubcoreMesh(core_axis_name="c", subcore_axis_name="s")


def _degree_body(col_hbm, zeros_hbm, out_hbm, ones_v, idx_v, ones_t, idx_t,
                 hist_s):
    c = lax.axis_index("c")
    s = lax.axis_index("s")
    wid = s * NC + c
    pltpu.sync_copy(zeros_hbm, hist_s.at[pl.ds(s * HSLICE, HSLICE)])
    for k in range(CHUNK // 16):
        ones_v[pl.ds(k * 16, 16)] = jnp.ones((16,), jnp.float32)
    ones_t[...] = jnp.ones((TAIL,), jnp.float32)
    plsc.subcore_barrier()

    base = wid * E_PER

    def chunk(i, carry):
        pltpu.sync_copy(col_hbm.at[pl.ds(base + i * CHUNK, CHUNK)], idx_v)
        pltpu.sync_copy(ones_v, hist_s.at[idx_v], add=True)
        return carry

    lax.fori_loop(0, NFULL, chunk, 0)
    pltpu.sync_copy(col_hbm.at[pl.ds(base + NFULL * CHUNK, TAIL)], idx_t)
    pltpu.sync_copy(ones_t, hist_s.at[idx_t], add=True)
    plsc.subcore_barrier()
    pltpu.sync_copy(hist_s.at[pl.ds(s * HSLICE, HSLICE)],
                    out_hbm.at[c, pl.ds(s * HSLICE, HSLICE)])


_sc_degree = pl.kernel(
    _degree_body,
    out_type=jax.ShapeDtypeStruct((NC, NPAD), jnp.float32),
    mesh=_MESH,
    scratch_types=[
        pltpu.VMEM((CHUNK,), jnp.float32),
        pltpu.VMEM((CHUNK,), jnp.int32),
        pltpu.VMEM((TAIL,), jnp.float32),
        pltpu.VMEM((TAIL,), jnp.int32),
        pltpu.VMEM_SHARED((NPAD,), jnp.float32),
    ],
)


def _scatter_body(y_hbm, row_hbm, col_hbm, zrows_hbm, out_hbm,
                  idx_r, idx_c, rows_v, idx_rt, idx_ct, rows_t, sem, acc_s):
    c = lax.axis_index("c")
    s = lax.axis_index("s")
    wid = s * NC + c
    pltpu.sync_copy(zrows_hbm, acc_s.at[pl.ds(s * RPT, RPT)])
    plsc.subcore_barrier()

    base = wid * E_PER

    def chunk(i, carry):
        pltpu.sync_copy(row_hbm.at[pl.ds(base + i * CHUNK, CHUNK)], idx_r)
        pltpu.sync_copy(col_hbm.at[pl.ds(base + i * CHUNK, CHUNK)], idx_c)
        pltpu.async_copy(y_hbm.at[idx_r], rows_v, sem).wait()
        pltpu.sync_copy(rows_v, acc_s.at[idx_c], add=True)
        return carry

    lax.fori_loop(0, NFULL, chunk, 0)
    pltpu.sync_copy(row_hbm.at[pl.ds(base + NFULL * CHUNK, TAIL)], idx_rt)
    pltpu.sync_copy(col_hbm.at[pl.ds(base + NFULL * CHUNK, TAIL)], idx_ct)
    pltpu.async_copy(y_hbm.at[idx_rt], rows_t, sem).wait()
    pltpu.sync_copy(rows_t, acc_s.at[idx_ct], add=True)
    plsc.subcore_barrier()
    pltpu.sync_copy(acc_s.at[pl.ds(s * RPT, RPT)],
                    out_hbm.at[c, pl.ds(s * RPT, RPT)])


_sc_scatter = pl.kernel(
    _scatter_body,
    out_type=jax.ShapeDtypeStruct((NC, NPAD, FDIM), jnp.float32),
    mesh=_MESH,
    scratch_types=[
        pltpu.VMEM((CHUNK,), jnp.int32),
        pltpu.VMEM((CHUNK,), jnp.int32),
        pltpu.VMEM((CHUNK, FDIM), jnp.float32),
        pltpu.VMEM((TAIL,), jnp.int32),
        pltpu.VMEM((TAIL,), jnp.int32),
        pltpu.VMEM((TAIL, FDIM), jnp.float32),
        pltpu.SemaphoreType.DMA,
        pltpu.VMEM_SHARED((NPAD, FDIM), jnp.float32),
    ],
)


def _dinv(deg_ref):
    deg = deg_ref[0, :N_NODES] + deg_ref[1, :N_NODES] + 1.0
    return lax.rsqrt(deg)[:, None]


def _prepare_body(x_ref, w1_ref, deg_ref, y1_ref):
    xw = jnp.dot(x_ref[...], w1_ref[...], preferred_element_type=jnp.float32)
    y1_ref[...] = xw * _dinv(deg_ref)


_tc_prepare = pl.pallas_call(
    _prepare_body,
    out_shape=jax.ShapeDtypeStruct((N_NODES, FDIM), jnp.float32),
)


def _mid_body(agg_ref, y1_ref, deg_ref, b1_ref, w2_ref, y2_ref):
    dinv = _dinv(deg_ref)
    pre = dinv * (agg_ref[0, :N_NODES] + agg_ref[1, :N_NODES] + y1_ref[...]) + b1_ref[...]
    h1 = jnp.maximum(pre, 0.0)
    y2_ref[...] = jnp.dot(h1, w2_ref[...],
                          preferred_element_type=jnp.float32) * dinv


_tc_mid = pl.pallas_call(
    _mid_body,
    out_shape=jax.ShapeDtypeStruct((N_NODES, FDIM), jnp.float32),
)


def _final_body(agg_ref, y2_ref, deg_ref, b2_ref, ws_ref, bs_ref,
                h2_ref, sc_ref):
    dinv = _dinv(deg_ref)
    pre = dinv * (agg_ref[0, :N_NODES] + agg_ref[1, :N_NODES] + y2_ref[...]) + b2_ref[...]
    h2 = jnp.maximum(pre, 0.0)
    h2_ref[...] = h2
    sc_ref[...] = jnp.dot(h2, ws_ref[...],
                          preferred_element_type=jnp.float32)[:, 0] + bs_ref[0]


_tc_final = pl.pallas_call(
    _final_body,
    out_shape=(
        jax.ShapeDtypeStruct((N_NODES, FDIM), jnp.float32),
        jax.ShapeDtypeStruct((N_NODES,), jnp.float32),
    ),
)


@jax.jit
def kernel(x, edge_index, W1, b1, W2, b2, Ws, bs):
    row = edge_index[0]
    col = edge_index[1]
    zeros_h = jnp.zeros((HSLICE,), jnp.float32)
    zrows = jnp.zeros((RPT, FDIM), jnp.float32)

    deg_part = _sc_degree(col, zeros_h)
    y1 = _tc_prepare(x, W1, deg_part)
    agg1 = _sc_scatter(y1, row, col, zrows)
    y2 = _tc_mid(agg1, y1, deg_part, b1, W2)
    agg2 = _sc_scatter(y2, row, col, zrows)
    h2, scores = _tc_final(agg2, y2, deg_part, b2, Ws, bs)
    return (h2, scores)


# speedup vs baseline: 23.7323x; 1.4196x over previous
import jax
import jax.numpy as jnp
from jax import lax
from jax.experimental import pallas as pl
from jax.experimental.pallas import tpu as pltpu
from jax.experimental.pallas import tpu_sc as plsc

N_NODES = 10000
N_EDGES = 320000
FDIM = 128

NC = 2
NS = 16
NW = NC * NS
CHUNK = 64
NCH = 160
E_PER = NCH * CHUNK
E_PAD = NW * E_PER
NPAD = 10240
HSLICE = NPAD // NS
RPT = NPAD // NS
NBUF = 2
NPAIR = NCH // NBUF

_MESH = plsc.VectorSubcoreMesh(core_axis_name="c", subcore_axis_name="s")


def _degree_body(col_hbm, zeros_hbm, out_hbm, ones_v, col_v, hist_s):
    c = lax.axis_index("c")
    s = lax.axis_index("s")
    wid = s * NC + c
    pltpu.sync_copy(zeros_hbm, hist_s.at[pl.ds(s * HSLICE, HSLICE)])
    pltpu.sync_copy(col_hbm.at[pl.ds(wid * NCH, NCH)], col_v)
    for k in range(CHUNK // 16):
        ones_v[pl.ds(k * 16, 16)] = jnp.ones((16,), jnp.float32)
    plsc.subcore_barrier()

    def chunk(q, carry):
        pltpu.sync_copy(ones_v, hist_s.at[col_v.at[q]], add=True)
        return carry

    lax.fori_loop(0, NCH, chunk, 0)
    plsc.subcore_barrier()
    pltpu.sync_copy(hist_s.at[pl.ds(s * HSLICE, HSLICE)],
                    out_hbm.at[c, pl.ds(s * HSLICE, HSLICE)])


_sc_degree = pl.kernel(
    _degree_body,
    out_type=jax.ShapeDtypeStruct((NC, NPAD), jnp.float32),
    mesh=_MESH,
    scratch_types=[
        pltpu.VMEM((CHUNK,), jnp.float32),
        pltpu.VMEM((NCH, CHUNK), jnp.int32),
        pltpu.VMEM_SHARED((NPAD,), jnp.float32),
    ],
)


def _scatter_body(y_hbm, row_hbm, col_hbm, zrows_hbm, out_hbm,
                  ridx0, ridx1, col_v, rows0, rows1,
                  sem0, sem1, acc_s):
    c = lax.axis_index("c")
    s = lax.axis_index("s")
    wid = s * NC + c
    rows = (rows0, rows1)
    sems = (sem0, sem1)
    ridx = (ridx0, ridx1)

    pltpu.sync_copy(zrows_hbm, acc_s.at[pl.ds(s * RPT, RPT)])
    pltpu.sync_copy(col_hbm.at[pl.ds(wid * NCH, NCH)], col_v)
    plsc.subcore_barrier()

    base = wid * NCH

    def load_gather(q, k):
        pltpu.sync_copy(row_hbm.at[base + q], ridx[k])
        pltpu.async_copy(y_hbm.at[ridx[k]], rows[k], sems[k])

    def wait_gather(k):
        pltpu.make_async_copy(y_hbm.at[ridx[k]], rows[k], sems[k]).wait()

    def scatter(q, k):
        pltpu.sync_copy(rows[k], acc_s.at[col_v.at[q]], add=True)

    load_gather(0, 0)
    load_gather(1, 1)

    def pair(j, carry):
        for k in range(NBUF):
            q = NBUF * j + k
            wait_gather(k)
            scatter(q, k)

            @pl.when(q + NBUF < NCH)
            def _():
                load_gather(q + NBUF, k)
        return carry

    lax.fori_loop(0, NPAIR, pair, 0)
    plsc.subcore_barrier()
    pltpu.sync_copy(acc_s.at[pl.ds(s * RPT, RPT)],
                    out_hbm.at[c, pl.ds(s * RPT, RPT)])


_sc_scatter = pl.kernel(
    _scatter_body,
    out_type=jax.ShapeDtypeStruct((NC, NPAD, FDIM), jnp.float32),
    mesh=_MESH,
    scratch_types=[
        pltpu.VMEM((CHUNK,), jnp.int32),
        pltpu.VMEM((CHUNK,), jnp.int32),
        pltpu.VMEM((NCH, CHUNK), jnp.int32),
        pltpu.VMEM((CHUNK, FDIM), jnp.float32),
        pltpu.VMEM((CHUNK, FDIM), jnp.float32),
        pltpu.SemaphoreType.DMA,
        pltpu.SemaphoreType.DMA,
        pltpu.VMEM_SHARED((NPAD, FDIM), jnp.float32),
    ],
)


def _dinv(deg_ref):
    deg = deg_ref[0, :N_NODES] + deg_ref[1, :N_NODES] + 1.0
    return lax.rsqrt(deg)[:, None]


def _prepare_body(x_ref, w1_ref, deg_ref, y1_ref):
    xw = jnp.dot(x_ref[...], w1_ref[...], preferred_element_type=jnp.float32)
    y1_ref[...] = xw * _dinv(deg_ref)


_tc_prepare = pl.pallas_call(
    _prepare_body,
    out_shape=jax.ShapeDtypeStruct((N_NODES, FDIM), jnp.float32),
)


def _mid_body(agg_ref, y1_ref, deg_ref, b1_ref, w2_ref, y2_ref):
    dinv = _dinv(deg_ref)
    pre = dinv * (agg_ref[0, :N_NODES] + agg_ref[1, :N_NODES]
                  + y1_ref[...]) + b1_ref[...]
    h1 = jnp.maximum(pre, 0.0)
    y2_ref[...] = jnp.dot(h1, w2_ref[...],
                          preferred_element_type=jnp.float32) * dinv


_tc_mid = pl.pallas_call(
    _mid_body,
    out_shape=jax.ShapeDtypeStruct((N_NODES, FDIM), jnp.float32),
)


def _final_body(agg_ref, y2_ref, deg_ref, b2_ref, ws_ref, bs_ref,
                h2_ref, sc_ref):
    dinv = _dinv(deg_ref)
    pre = dinv * (agg_ref[0, :N_NODES] + agg_ref[1, :N_NODES]
                  + y2_ref[...]) + b2_ref[...]
    h2 = jnp.maximum(pre, 0.0)
    h2_ref[...] = h2
    sc_ref[...] = jnp.dot(h2, ws_ref[...],
                          preferred_element_type=jnp.float32)[:, 0] + bs_ref[0]


_tc_final = pl.pallas_call(
    _final_body,
    out_shape=(
        jax.ShapeDtypeStruct((N_NODES, FDIM), jnp.float32),
        jax.ShapeDtypeStruct((N_NODES,), jnp.float32),
    ),
)


@jax.jit
def kernel(x, edge_index, W1, b1, W2, b2, Ws, bs):
    npad_e = E_PAD - N_EDGES
    pad_row = (jnp.arange(npad_e, dtype=jnp.int32) * 37) % N_NODES
    pad_col = N_NODES + (jnp.arange(npad_e, dtype=jnp.int32) % (NPAD - N_NODES))
    row2d = jnp.concatenate([edge_index[0], pad_row]).reshape(NW * NCH, CHUNK)
    col2d = jnp.concatenate([edge_index[1], pad_col]).reshape(NW * NCH, CHUNK)
    zeros_h = jnp.zeros((HSLICE,), jnp.float32)
    zrows = jnp.zeros((RPT, FDIM), jnp.float32)

    deg_part = _sc_degree(col2d, zeros_h)
    y1 = _tc_prepare(x, W1, deg_part)
    agg1 = _sc_scatter(y1, row2d, col2d, zrows)
    y2 = _tc_mid(agg1, y1, deg_part, b1, W2)
    agg2 = _sc_scatter(y2, row2d, col2d, zrows)
    h2, scores = _tc_final(agg2, y2, deg_part, b2, Ws, bs)
    return (h2, scores)


# speedup vs baseline: 34.0570x; 1.4351x over previous
import jax
import jax.numpy as jnp
from jax import lax
from jax.experimental import pallas as pl
from jax.experimental.pallas import tpu as pltpu
from jax.experimental.pallas import tpu_sc as plsc

N_NODES = 10000
N_EDGES = 320000
FDIM = 128

NC = 2
NS = 16
NW = NC * NS
CHUNK = 128
NCH = 80
E_PER = NCH * CHUNK
E_PAD = NW * E_PER
NPAD = 10240
HSLICE = NPAD // NS
RPT = NPAD // NS
NBUF = 2
BS = 8
NBAT = NCH // BS

_MESH = plsc.VectorSubcoreMesh(core_axis_name="c", subcore_axis_name="s")


def _degree_body(col_hbm, zeros_hbm, out_hbm, ones_v, col_v, hist_s):
    c = lax.axis_index("c")
    s = lax.axis_index("s")
    wid = s * NC + c
    pltpu.sync_copy(zeros_hbm, hist_s.at[pl.ds(s * HSLICE, HSLICE)])
    pltpu.sync_copy(col_hbm.at[pl.ds(wid * NCH, NCH)], col_v)
    for k in range(CHUNK // 16):
        ones_v[pl.ds(k * 16, 16)] = jnp.ones((16,), jnp.float32)
    plsc.subcore_barrier()

    def chunk(q, carry):
        pltpu.sync_copy(ones_v, hist_s.at[col_v.at[q]], add=True)
        return carry

    lax.fori_loop(0, NCH, chunk, 0)
    plsc.subcore_barrier()
    pltpu.sync_copy(hist_s.at[pl.ds(s * HSLICE, HSLICE)],
                    out_hbm.at[c, pl.ds(s * HSLICE, HSLICE)])


_sc_degree = pl.kernel(
    _degree_body,
    out_type=jax.ShapeDtypeStruct((NC, NPAD), jnp.float32),
    mesh=_MESH,
    scratch_types=[
        pltpu.VMEM((CHUNK,), jnp.float32),
        pltpu.VMEM((NCH, CHUNK), jnp.int32),
        pltpu.VMEM_SHARED((NPAD,), jnp.float32),
    ],
)


def _scatter_body(y_hbm, row_hbm, col_hbm, zrows_hbm, out_hbm,
                  ridxa, ridxb, isema, isemb, col_v, rows0, rows1,
                  sem0, sem1, acc_s):
    c = lax.axis_index("c")
    s = lax.axis_index("s")
    wid = s * NC + c
    rows = (rows0, rows1)
    sems = (sem0, sem1)

    pltpu.sync_copy(zrows_hbm, acc_s.at[pl.ds(s * RPT, RPT)])
    pltpu.sync_copy(col_hbm.at[pl.ds(wid * NCH, NCH)], col_v)
    plsc.subcore_barrier()

    base = wid * NCH

    def load_batch(b, rbuf, rsem):
        pltpu.async_copy(row_hbm.at[pl.ds(base + b * BS, BS)], rbuf, rsem)

    def wait_batch(b, rbuf, rsem):
        pltpu.make_async_copy(row_hbm.at[pl.ds(base + b * BS, BS)],
                              rbuf, rsem).wait()

    def gather(rbuf, t, k):
        pltpu.async_copy(y_hbm.at[rbuf.at[t]], rows[k], sems[k])

    def wait_gather(rbuf, t, k):
        pltpu.make_async_copy(y_hbm.at[rbuf.at[t]], rows[k], sems[k]).wait()

    def scatter(q, k):
        pltpu.sync_copy(rows[k], acc_s.at[col_v.at[q]], add=True)

    load_batch(0, ridxa, isema)
    wait_batch(0, ridxa, isema)
    load_batch(1, ridxb, isemb)
    gather(ridxa, 0, 0)
    gather(ridxa, 1, 1)

    def super_step(m, carry):
        for t in range(2 * BS):
            q = 2 * BS * m + t
            k = t % 2
            wait_gather(ridxa if t < BS else ridxb, t % BS, k)
            scatter(q, k)
            nt = t + 2
            if nt < BS:
                gather(ridxa, nt, k)
            elif nt < 2 * BS:
                if nt == BS:

                    @pl.when(2 * m + 1 < NBAT)
                    def _():
                        wait_batch(2 * m + 1, ridxb, isemb)

                gather(ridxb, nt - BS, k)
            else:

                @pl.when(q + 2 < NCH)
                def _():
                    if nt == 2 * BS:

                        @pl.when(2 * m + 2 < NBAT)
                        def _():
                            wait_batch(2 * m + 2, ridxa, isema)

                    gather(ridxa, nt - 2 * BS, k)

            if t == BS - 1:

                @pl.when(2 * m + 2 < NBAT)
                def _():
                    load_batch(2 * m + 2, ridxa, isema)

            if t == 2 * BS - 1:

                @pl.when(2 * m + 3 < NBAT)
                def _():
                    load_batch(2 * m + 3, ridxb, isemb)

        return carry

    lax.fori_loop(0, NBAT // 2, super_step, 0)
    plsc.subcore_barrier()
    pltpu.sync_copy(acc_s.at[pl.ds(s * RPT, RPT)],
                    out_hbm.at[c, pl.ds(s * RPT, RPT)])


_sc_scatter = pl.kernel(
    _scatter_body,
    out_type=jax.ShapeDtypeStruct((NC, NPAD, FDIM), jnp.float32),
    mesh=_MESH,
    scratch_types=[
        pltpu.VMEM((BS, CHUNK), jnp.int32),
        pltpu.VMEM((BS, CHUNK), jnp.int32),
        pltpu.SemaphoreType.DMA,
        pltpu.SemaphoreType.DMA,
        pltpu.VMEM((NCH, CHUNK), jnp.int32),
        pltpu.VMEM((CHUNK, FDIM), jnp.float32),
        pltpu.VMEM((CHUNK, FDIM), jnp.float32),
        pltpu.SemaphoreType.DMA,
        pltpu.SemaphoreType.DMA,
        pltpu.VMEM_SHARED((NPAD, FDIM), jnp.float32),
    ],
)


def _dinv(deg_ref):
    deg = deg_ref[0, :N_NODES] + deg_ref[1, :N_NODES] + 1.0
    return lax.rsqrt(deg)[:, None]


def _prepare_body(x_ref, w1_ref, deg_ref, y1_ref):
    xw = jnp.dot(x_ref[...], w1_ref[...], preferred_element_type=jnp.float32)
    y1_ref[...] = xw * _dinv(deg_ref)


_tc_prepare = pl.pallas_call(
    _prepare_body,
    out_shape=jax.ShapeDtypeStruct((N_NODES, FDIM), jnp.float32),
)


def _mid_body(agg_ref, y1_ref, deg_ref, b1_ref, w2_ref, y2_ref):
    dinv = _dinv(deg_ref)
    pre = dinv * (agg_ref[0, :N_NODES] + agg_ref[1, :N_NODES]
                  + y1_ref[...]) + b1_ref[...]
    h1 = jnp.maximum(pre, 0.0)
    y2_ref[...] = jnp.dot(h1, w2_ref[...],
                          preferred_element_type=jnp.float32) * dinv


_tc_mid = pl.pallas_call(
    _mid_body,
    out_shape=jax.ShapeDtypeStruct((N_NODES, FDIM), jnp.float32),
)


def _final_body(agg_ref, y2_ref, deg_ref, b2_ref, ws_ref, bs_ref,
                h2_ref, sc_ref):
    dinv = _dinv(deg_ref)
    pre = dinv * (agg_ref[0, :N_NODES] + agg_ref[1, :N_NODES]
                  + y2_ref[...]) + b2_ref[...]
    h2 = jnp.maximum(pre, 0.0)
    h2_ref[...] = h2
    sc_ref[...] = jnp.dot(h2, ws_ref[...],
                          preferred_element_type=jnp.float32)[:, 0] + bs_ref[0]


_tc_final = pl.pallas_call(
    _final_body,
    out_shape=(
        jax.ShapeDtypeStruct((N_NODES, FDIM), jnp.float32),
        jax.ShapeDtypeStruct((N_NODES,), jnp.float32),
    ),
)


@jax.jit
def kernel(x, edge_index, W1, b1, W2, b2, Ws, bs):
    npad_e = E_PAD - N_EDGES
    pad_row = (jnp.arange(npad_e, dtype=jnp.int32) * 37) % N_NODES
    pad_col = N_NODES + (jnp.arange(npad_e, dtype=jnp.int32) % (NPAD - N_NODES))
    row2d = jnp.concatenate([edge_index[0], pad_row]).reshape(NW * NCH, CHUNK)
    col2d = jnp.concatenate([edge_index[1], pad_col]).reshape(NW * NCH, CHUNK)
    zeros_h = jnp.zeros((HSLICE,), jnp.float32)
    zrows = jnp.zeros((RPT, FDIM), jnp.float32)

    deg_part = _sc_degree(col2d, zeros_h)
    y1 = _tc_prepare(x, W1, deg_part)
    agg1 = _sc_scatter(y1, row2d, col2d, zrows)
    y2 = _tc_mid(agg1, y1, deg_part, b1, W2)
    agg2 = _sc_scatter(y2, row2d, col2d, zrows)
    h2, scores = _tc_final(agg2, y2, deg_part, b2, Ws, bs)
    return (h2, scores)


# speedup vs baseline: 34.0840x; 1.0008x over previous
import jax
import jax.numpy as jnp
import numpy as np
from jax import lax
from jax.experimental import pallas as pl
from jax.experimental.pallas import tpu as pltpu
from jax.experimental.pallas import tpu_sc as plsc

N_NODES = 10000
N_EDGES = 320000
FDIM = 128

NC = 2
NS = 16
NW = NC * NS
CHUNK = 128
NCH = 80
E_PER = NCH * CHUNK
E_PAD = NW * E_PER
NPAD = 10240
HSLICE = NPAD // NS
RPT = NPAD // NS
NBUF = 2
BS = 8
NBAT = NCH // BS

_MESH = plsc.VectorSubcoreMesh(core_axis_name="c", subcore_axis_name="s")


def _degree_body(col_hbm, zeros_hbm, out_hbm, ones_v, col_v, hist_s):
    c = lax.axis_index("c")
    s = lax.axis_index("s")
    wid = s * NC + c
    pltpu.sync_copy(zeros_hbm, hist_s.at[pl.ds(s * HSLICE, HSLICE)])
    pltpu.sync_copy(col_hbm.at[pl.ds(wid * NCH, NCH)], col_v)
    for k in range(CHUNK // 16):
        ones_v[pl.ds(k * 16, 16)] = jnp.ones((16,), jnp.float32)
    plsc.subcore_barrier()

    def chunk(q, carry):
        pltpu.sync_copy(ones_v, hist_s.at[col_v.at[q]], add=True)
        return carry

    lax.fori_loop(0, NCH, chunk, 0)
    plsc.subcore_barrier()
    pltpu.sync_copy(hist_s.at[pl.ds(s * HSLICE, HSLICE)],
                    out_hbm.at[c, pl.ds(s * HSLICE, HSLICE)])


_sc_degree = pl.kernel(
    _degree_body,
    out_type=jax.ShapeDtypeStruct((NC, NPAD), jnp.float32),
    mesh=_MESH,
    scratch_types=[
        pltpu.VMEM((CHUNK,), jnp.float32),
        pltpu.VMEM((NCH, CHUNK), jnp.int32),
        pltpu.VMEM_SHARED((NPAD,), jnp.float32),
    ],
)


def _scatter_body(y_hbm, row_hbm, col_hbm, zrows_hbm, out_hbm,
                  ridxa, ridxb, isema, isemb, col_v, rows0, rows1,
                  sem0, sem1, acc_s):
    c = lax.axis_index("c")
    s = lax.axis_index("s")
    wid = s * NC + c
    rows = (rows0, rows1)
    sems = (sem0, sem1)

    pltpu.sync_copy(zrows_hbm, acc_s.at[pl.ds(s * RPT, RPT)])
    pltpu.sync_copy(col_hbm.at[pl.ds(wid * NCH, NCH)], col_v)
    plsc.subcore_barrier()

    base = wid * NCH

    def load_batch(b, rbuf, rsem):
        pltpu.async_copy(row_hbm.at[pl.ds(base + b * BS, BS)], rbuf, rsem)

    def wait_batch(b, rbuf, rsem):
        pltpu.make_async_copy(row_hbm.at[pl.ds(base + b * BS, BS)],
                              rbuf, rsem).wait()

    def gather(rbuf, t, k):
        pltpu.async_copy(y_hbm.at[rbuf.at[t]], rows[k], sems[k])

    def wait_gather(rbuf, t, k):
        pltpu.make_async_copy(y_hbm.at[rbuf.at[t]], rows[k], sems[k]).wait()

    def scatter(q, k):
        pltpu.sync_copy(rows[k], acc_s.at[col_v.at[q]], add=True)

    load_batch(0, ridxa, isema)
    wait_batch(0, ridxa, isema)
    load_batch(1, ridxb, isemb)
    gather(ridxa, 0, 0)
    gather(ridxa, 1, 1)

    def super_step(m, carry):
        for t in range(2 * BS):
            q = 2 * BS * m + t
            k = t % 2
            wait_gather(ridxa if t < BS else ridxb, t % BS, k)
            scatter(q, k)
            nt = t + 2
            if nt < BS:
                gather(ridxa, nt, k)
            elif nt < 2 * BS:
                if nt == BS:

                    @pl.when(2 * m + 1 < NBAT)
                    def _():
                        wait_batch(2 * m + 1, ridxb, isemb)

                gather(ridxb, nt - BS, k)
            else:

                @pl.when(q + 2 < NCH)
                def _():
                    if nt == 2 * BS:

                        @pl.when(2 * m + 2 < NBAT)
                        def _():
                            wait_batch(2 * m + 2, ridxa, isema)

                    gather(ridxa, nt - 2 * BS, k)

            if t == BS - 1:

                @pl.when(2 * m + 2 < NBAT)
                def _():
                    load_batch(2 * m + 2, ridxa, isema)

            if t == 2 * BS - 1:

                @pl.when(2 * m + 3 < NBAT)
                def _():
                    load_batch(2 * m + 3, ridxb, isemb)

        return carry

    lax.fori_loop(0, NBAT // 2, super_step, 0)
    plsc.subcore_barrier()
    pltpu.sync_copy(acc_s.at[pl.ds(s * RPT, RPT)],
                    out_hbm.at[c, pl.ds(s * RPT, RPT)])


_sc_scatter = pl.kernel(
    _scatter_body,
    out_type=jax.ShapeDtypeStruct((NC, NPAD, FDIM), jnp.float32),
    mesh=_MESH,
    scratch_types=[
        pltpu.VMEM((BS, CHUNK), jnp.int32),
        pltpu.VMEM((BS, CHUNK), jnp.int32),
        pltpu.SemaphoreType.DMA,
        pltpu.SemaphoreType.DMA,
        pltpu.VMEM((NCH, CHUNK), jnp.int32),
        pltpu.VMEM((CHUNK, FDIM), jnp.float32),
        pltpu.VMEM((CHUNK, FDIM), jnp.float32),
        pltpu.SemaphoreType.DMA,
        pltpu.SemaphoreType.DMA,
        pltpu.VMEM_SHARED((NPAD, FDIM), jnp.float32),
    ],
)


def _dinv(deg_ref):
    deg = deg_ref[0, :N_NODES] + deg_ref[1, :N_NODES] + 1.0
    return lax.rsqrt(deg)[:, None]


def _prepare_body(x_ref, w1_ref, deg_ref, y1_ref):
    xw = jnp.dot(x_ref[...], w1_ref[...], preferred_element_type=jnp.float32)
    y1_ref[...] = xw * _dinv(deg_ref)


_tc_prepare = pl.pallas_call(
    _prepare_body,
    out_shape=jax.ShapeDtypeStruct((N_NODES, FDIM), jnp.float32),
)


def _mid_body(agg_ref, y1_ref, deg_ref, b1_ref, w2_ref, y2_ref):
    dinv = _dinv(deg_ref)
    pre = dinv * (agg_ref[0, :N_NODES] + agg_ref[1, :N_NODES]
                  + y1_ref[...]) + b1_ref[...]
    h1 = jnp.maximum(pre, 0.0)
    y2_ref[...] = jnp.dot(h1, w2_ref[...],
                          preferred_element_type=jnp.float32) * dinv


_tc_mid = pl.pallas_call(
    _mid_body,
    out_shape=jax.ShapeDtypeStruct((N_NODES, FDIM), jnp.float32),
)


def _final_body(agg_ref, y2_ref, deg_ref, b2_ref, ws_ref, bs_ref,
                h2_ref, sc_ref):
    dinv = _dinv(deg_ref)
    pre = dinv * (agg_ref[0, :N_NODES] + agg_ref[1, :N_NODES]
                  + y2_ref[...]) + b2_ref[...]
    h2 = jnp.maximum(pre, 0.0)
    h2_ref[...] = h2
    sc_ref[...] = jnp.dot(h2, ws_ref[...],
                          preferred_element_type=jnp.float32)[:, 0] + bs_ref[0]


_tc_final = pl.pallas_call(
    _final_body,
    out_shape=(
        jax.ShapeDtypeStruct((N_NODES, FDIM), jnp.float32),
        jax.ShapeDtypeStruct((N_NODES,), jnp.float32),
    ),
)


_NPAD_E = E_PAD - N_EDGES
_PAD_ROW = np.asarray((np.arange(_NPAD_E) * 37) % N_NODES, dtype=np.int32)
_PAD_COL = np.asarray(N_NODES + np.arange(_NPAD_E) % (NPAD - N_NODES),
                      dtype=np.int32)


@jax.jit
def kernel(x, edge_index, W1, b1, W2, b2, Ws, bs):
    row2d = jnp.concatenate([edge_index[0], _PAD_ROW]).reshape(NW * NCH, CHUNK)
    col2d = jnp.concatenate([edge_index[1], _PAD_COL]).reshape(NW * NCH, CHUNK)
    zeros_h = jnp.zeros((HSLICE,), jnp.float32)
    zrows = jnp.zeros((RPT, FDIM), jnp.float32)

    deg_part = _sc_degree(col2d, zeros_h)
    y1 = _tc_prepare(x, W1, deg_part)
    agg1 = _sc_scatter(y1, row2d, col2d, zrows)
    y2 = _tc_mid(agg1, y1, deg_part, b1, W2)
    agg2 = _sc_scatter(y2, row2d, col2d, zrows)
    h2, scores = _tc_final(agg2, y2, deg_part, b2, Ws, bs)
    return (h2, scores)
